# Initial kernel scaffold; baseline (speedup 1.0000x reference)
import functools

import jax
import jax.numpy as jnp
from jax import lax
from jax.experimental import pallas as pl
from jax.experimental.pallas import tpu as pltpu
from jax.experimental.pallas import tpu_sc as plsc

N = 10000
E = 160000
IN_CH = 256
MID_CH = 512

NC = 2
NS = 16
LANES = 16

NP = 10240
HALF = NP // NC

DEG_STRIPE = E // (NC * NS)
DEG_CAP = 5120
DEG_CHUNKS = DEG_CAP // 128
N_DEG = 10016
DEG_DUMP = 10000
DEG_ZROWS = N_DEG // NS

_deg_mesh = plsc.VectorSubcoreMesh(core_axis_name="c", subcore_axis_name="s")


@functools.partial(
    pl.kernel,
    out_type=jax.ShapeDtypeStruct((NC, N_DEG, LANES), jnp.float32),
    mesh=_deg_mesh,
    scratch_types=[
        pltpu.VMEM((DEG_CAP,), jnp.int32),
        pltpu.VMEM((DEG_CHUNKS, 128), jnp.int32),
        pltpu.VMEM((128, LANES), jnp.float32),
        pltpu.VMEM_SHARED((N_DEG, LANES), jnp.float32),
    ],
)
def _deg_kernel(dst_hbm, zeros_hbm, ones_hbm, deg_out, dstbuf, cidx, ones_v,
                deg_sp):
    c = lax.axis_index("c")
    s = lax.axis_index("s")
    wid = s * NC + c

    pltpu.sync_copy(dst_hbm.at[pl.ds(wid * DEG_STRIPE, DEG_STRIPE)],
                    dstbuf.at[pl.ds(0, DEG_STRIPE)])
    pltpu.sync_copy(ones_hbm, ones_v)
    pltpu.sync_copy(zeros_hbm, deg_sp.at[pl.ds(s * DEG_ZROWS, DEG_ZROWS)])

    lane = lax.iota(jnp.int32, 16)

    def build(j, carry):
        for k in range(8):
            off = j * 128 + k * 16
            pos = off + lane
            v = dstbuf[pl.ds(off, 16)]
            valid = pos < DEG_STRIPE
            cv = jnp.where(valid, v, DEG_DUMP)
            cidx[j, pl.ds(k * 16, 16)] = cv
        return carry

    lax.fori_loop(0, DEG_CHUNKS, build, 0)

    plsc.subcore_barrier()

    def scat(j, carry):
        pltpu.sync_copy(ones_v, deg_sp.at[cidx.at[j]], add=True)
        return carry

    lax.fori_loop(0, DEG_CHUNKS, scat, 0)

    plsc.subcore_barrier()

    pltpu.sync_copy(deg_sp.at[pl.ds(s * DEG_ZROWS, DEG_ZROWS)],
                    deg_out.at[c, pl.ds(s * DEG_ZROWS, DEG_ZROWS)])


EDGE_STRIPE = E // NS
EDGE_CAP = 10112
EDGE_CHUNKS = EDGE_CAP // 128
ACC_ROWS = HALF + 8
ACC_DUMP = HALF
ACC_ZROWS = HALF // NS

_acc_mesh = plsc.VectorSubcoreMesh(core_axis_name="c", subcore_axis_name="s")


@functools.partial(
    pl.kernel,
    out_type=jax.ShapeDtypeStruct((NC, HALF, IN_CH), jnp.float32),
    mesh=_acc_mesh,
    scratch_types=[
        pltpu.VMEM((EDGE_CAP,), jnp.int32),
        pltpu.VMEM((EDGE_CAP,), jnp.int32),
        pltpu.VMEM((EDGE_CAP,), jnp.int32),
        pltpu.VMEM((EDGE_CHUNKS, 128), jnp.int32),
        pltpu.VMEM((128, IN_CH), jnp.float32),
        pltpu.VMEM_SHARED((ACC_ROWS, IN_CH), jnp.float32),
    ],
)
def _acc_kernel(y_hbm, src_hbm, dst_hbm, zeros_hbm, acc_out,
                srcbuf, dstbuf, csrc, cdst, rows, acc_sp):
    c = lax.axis_index("c")
    s = lax.axis_index("s")
    lo = c * HALF

    pltpu.sync_copy(src_hbm.at[pl.ds(s * EDGE_STRIPE, EDGE_STRIPE)],
                    srcbuf.at[pl.ds(0, EDGE_STRIPE)])
    pltpu.sync_copy(dst_hbm.at[pl.ds(s * EDGE_STRIPE, EDGE_STRIPE)],
                    dstbuf.at[pl.ds(0, EDGE_STRIPE)])
    pltpu.sync_copy(zeros_hbm, acc_sp.at[pl.ds(s * ACC_ZROWS, ACC_ZROWS)])

    lane = lax.iota(jnp.int32, 16)

    def build(j, carry):
        for k in range(8):
            off = j * 128 + k * 16
            pos = off + lane
            vs = srcbuf[pl.ds(off, 16)]
            vd = dstbuf[pl.ds(off, 16)]
            valid = pos < EDGE_STRIPE
            mine = valid & (vd >= lo) & (vd < lo + HALF)
            csrc[pl.ds(off, 16)] = jnp.where(valid, vs, 0)
            cdst[j, pl.ds(k * 16, 16)] = jnp.where(mine, vd - lo, ACC_DUMP)
        return carry

    lax.fori_loop(0, EDGE_CHUNKS, build, 0)

    plsc.subcore_barrier()

    def edge_chunk(j, carry):
        pltpu.sync_copy(y_hbm.at[csrc.at[pl.ds(j * 128, 128)]], rows)
        pltpu.sync_copy(rows, acc_sp.at[cdst.at[j]], add=True)
        return carry

    lax.fori_loop(0, EDGE_CHUNKS, edge_chunk, 0)

    plsc.subcore_barrier()

    pltpu.sync_copy(acc_sp.at[pl.ds(s * ACC_ZROWS, ACC_ZROWS)],
                    acc_out.at[c, pl.ds(s * ACC_ZROWS, ACC_ZROWS)])


RB = 512
GRID = NP // RB


def _pre_body(x_ref, w_ref, p0_ref, p1_ref, y_ref, dinv_ref):
    xw = jnp.dot(x_ref[...], w_ref[...], preferred_element_type=jnp.float32)
    deg = 1.0 + p0_ref[...] + p1_ref[...]
    dinv = lax.rsqrt(deg)
    dinv_ref[...] = dinv
    y_ref[...] = xw * dinv


_pre_call = pl.pallas_call(
    _pre_body,
    grid=(GRID,),
    in_specs=[
        pl.BlockSpec((RB, IN_CH), lambda i: (i, 0)),
        pl.BlockSpec((IN_CH, IN_CH), lambda i: (0, 0)),
        pl.BlockSpec((RB, 1), lambda i: (i, 0)),
        pl.BlockSpec((RB, 1), lambda i: (i, 0)),
    ],
    out_specs=[
        pl.BlockSpec((RB, IN_CH), lambda i: (i, 0)),
        pl.BlockSpec((RB, 1), lambda i: (i, 0)),
    ],
    out_shape=[
        jax.ShapeDtypeStruct((NP, IN_CH), jnp.float32),
        jax.ShapeDtypeStruct((NP, 1), jnp.float32),
    ],
)


def _head_body(acc_ref, y_ref, dinv_ref, x_ref, bg_ref, w1_ref, b1_ref,
               w2_ref, b2_ref, w3_ref, b3_ref, x0_ref, x1_ref):
    gcn = (acc_ref[...] + y_ref[...]) * dinv_ref[...] + bg_ref[...]
    h = jnp.maximum(gcn, 0.0) + x_ref[...]
    x0 = jnp.maximum(
        jnp.dot(h, w1_ref[...], preferred_element_type=jnp.float32)
        + b1_ref[...], 0.0)
    x0_ref[...] = x0
    x1h = jnp.maximum(
        jnp.dot(x0, w2_ref[...], preferred_element_type=jnp.float32)
        + b2_ref[...], 0.0)
    x1_ref[...] = (jnp.dot(x1h, w3_ref[...],
                           preferred_element_type=jnp.float32) + b3_ref[...])


_head_call = pl.pallas_call(
    _head_body,
    grid=(GRID,),
    in_specs=[
        pl.BlockSpec((RB, IN_CH), lambda i: (i, 0)),
        pl.BlockSpec((RB, IN_CH), lambda i: (i, 0)),
        pl.BlockSpec((RB, 1), lambda i: (i, 0)),
        pl.BlockSpec((RB, IN_CH), lambda i: (i, 0)),
        pl.BlockSpec((1, IN_CH), lambda i: (0, 0)),
        pl.BlockSpec((IN_CH, MID_CH), lambda i: (0, 0)),
        pl.BlockSpec((1, MID_CH), lambda i: (0, 0)),
        pl.BlockSpec((MID_CH, MID_CH), lambda i: (0, 0)),
        pl.BlockSpec((1, MID_CH), lambda i: (0, 0)),
        pl.BlockSpec((MID_CH, 128), lambda i: (0, 0)),
        pl.BlockSpec((1, 128), lambda i: (0, 0)),
    ],
    out_specs=[
        pl.BlockSpec((RB, MID_CH), lambda i: (i, 0)),
        pl.BlockSpec((RB, 128), lambda i: (i, 0)),
    ],
    out_shape=[
        jax.ShapeDtypeStruct((NP, MID_CH), jnp.float32),
        jax.ShapeDtypeStruct((NP, 128), jnp.float32),
    ],
)


@jax.jit
def kernel(x, edge_index, W_gcn, b_gcn, W1, b1, W2, b2, W3, b3):
    ei = edge_index.astype(jnp.int32)
    src = ei[0]
    dst = ei[1]

    x_p = jnp.pad(x, ((0, NP - N), (0, 0)))

    deg_parts = _deg_kernel(
        dst,
        jnp.zeros((DEG_ZROWS, LANES), jnp.float32),
        jnp.ones((128, LANES), jnp.float32),
    )
    p0 = jnp.pad(deg_parts[0, :N, 0:1], ((0, NP - N), (0, 0)))
    p1 = jnp.pad(deg_parts[1, :N, 0:1], ((0, NP - N), (0, 0)))

    y, dinv = _pre_call(x_p, W_gcn, p0, p1)

    acc = _acc_kernel(y, src, dst,
                      jnp.zeros((ACC_ZROWS, IN_CH), jnp.float32))
    acc = acc.reshape(NP, IN_CH)

    x0, x1 = _head_call(
        acc, y, dinv, x_p,
        b_gcn.reshape(1, IN_CH),
        W1, b1.reshape(1, MID_CH),
        W2, b2.reshape(1, MID_CH),
        jnp.pad(W3, ((0, 0), (0, 127))),
        jnp.pad(b3.reshape(1, 1), ((0, 0), (0, 127))),
    )
    return x0[:N], x1[:N, 0:1]


# speedup vs baseline: 2.0847x; 2.0847x over previous
import functools

import jax
import jax.numpy as jnp
from jax import lax
from jax.experimental import pallas as pl
from jax.experimental.pallas import tpu as pltpu
from jax.experimental.pallas import tpu_sc as plsc

N = 10000
E = 160000
IN_CH = 256
MID_CH = 512
HC = 128

NC = 2
NS = 16
NT = NC * NS
NP = 10240
OWN = NP // NS
ACCR = OWN + 8
DUMP = OWN

STRIPE = E // NT
SPAD = 5120
SEG = 6144
CW = 1024

_mesh = plsc.VectorSubcoreMesh(core_axis_name="c", subcore_axis_name="s")


@functools.partial(
    pl.kernel,
    out_type=(jax.ShapeDtypeStruct((NS * NT * SEG,), jnp.int32),
              jax.ShapeDtypeStruct((NS * NT * CW,), jnp.int32)),
    mesh=_mesh,
    scratch_types=[
        pltpu.VMEM((SPAD,), jnp.int32),
        pltpu.VMEM((SPAD,), jnp.int32),
        pltpu.VMEM((NS * SEG,), jnp.int32),
        pltpu.VMEM((NS * 16,), jnp.int32),
        pltpu.VMEM((CW,), jnp.int32),
    ],
)
def _route_kernel(src_hbm, dst_hbm, routed, counts, sbuf, dbuf, bpk, cnt16,
                  cpad):
    c = lax.axis_index("c")
    s = lax.axis_index("s")
    wid = s * NC + c

    pltpu.sync_copy(src_hbm.at[pl.ds(wid * SPAD, SPAD)], sbuf)
    pltpu.sync_copy(dst_hbm.at[pl.ds(wid * SPAD, SPAD)], dbuf)

    zero16 = jnp.zeros((16,), jnp.int32)
    for o in range(NS):
        cnt16[pl.ds(o * 16, 16)] = zero16

    def scan(i, carry):
        vs = sbuf[pl.ds(i * 16, 16)]
        vd = dbuf[pl.ds(i * 16, 16)]
        vo = jnp.minimum((vd * 6554) >> 22, NS - 1)
        vdl = vd - vo * OWN
        vpk = vs * 1024 + vdl
        for l in range(16):
            o = vo[l]
            cnt = cnt16[pl.ds(o * 16, 16)][0]
            bpk[pl.ds(o * SEG + cnt, 16)] = jnp.full((16,), vpk[l],
                                                     jnp.int32)
            cnt16[pl.ds(o * 16, 16)] = jnp.full((16,), cnt + 1, jnp.int32)
        return carry

    lax.fori_loop(0, SPAD // 16, scan, 0)

    pad = jnp.full((16,), DUMP, jnp.int32)
    for o in range(NS):
        cnt = cnt16[pl.ds(o * 16, 16)][0]
        for k in range(9):
            bpk[pl.ds(o * SEG + cnt + k * 16, 16)] = pad

    for o in range(NS):
        pltpu.sync_copy(bpk.at[pl.ds(o * SEG, SEG)],
                        routed.at[pl.ds((o * NT + wid) * SEG, SEG)])
        cpad[pl.ds(0, 16)] = cnt16[pl.ds(o * 16, 16)]
        pltpu.sync_copy(cpad, counts.at[pl.ds((o * NT + wid) * CW, CW)])


DEGSEG = OWN * 16


@functools.partial(
    pl.kernel,
    out_type=jax.ShapeDtypeStruct((NC * NS * DEGSEG,), jnp.float32),
    mesh=_mesh,
    scratch_types=[
        pltpu.VMEM((SEG + 16,), jnp.int32),
        pltpu.VMEM((CW,), jnp.int32),
        pltpu.VMEM((ACCR * 16,), jnp.float32),
    ],
)
def _deg_kernel(routed_hbm, counts_hbm, zeros_hbm, deg_out, pkbuf, cbuf,
                hist):
    c = lax.axis_index("c")
    s = lax.axis_index("s")

    pltpu.sync_copy(zeros_hbm, hist)

    ones16 = jnp.full((16,), 1.0, jnp.float32)

    def producer(w2h, carry):
        w2 = w2h * NC + c
        pltpu.sync_copy(routed_hbm.at[pl.ds((s * NT + w2) * SEG, SEG)],
                        pkbuf.at[pl.ds(0, SEG)])
        pltpu.sync_copy(counts_hbm.at[pl.ds((s * NT + w2) * CW, CW)], cbuf)
        cnt = cbuf[pl.ds(0, 16)][0]
        nb = (cnt + 127) // 128

        def chunk(b, carry2):
            def edge(e, carry3):
                dl = pkbuf[pl.ds(b * 128 + e, 16)][0] & 1023
                plsc.addupdate(hist.at[pl.ds(dl * 16, 16)], ones16)
                return carry3
            lax.fori_loop(0, 128, edge, 0)
            return carry2

        lax.fori_loop(0, nb, chunk, 0)
        return carry

    lax.fori_loop(0, NT // NC, producer, 0)

    pltpu.sync_copy(hist.at[pl.ds(0, DEGSEG)],
                    deg_out.at[pl.ds((c * NS + s) * DEGSEG, DEGSEG)])


@functools.partial(
    pl.kernel,
    out_type=jax.ShapeDtypeStruct((NP, IN_CH), jnp.float32),
    mesh=_mesh,
    scratch_types=[
        pltpu.VMEM((SEG + 16,), jnp.int32),
        pltpu.VMEM((CW,), jnp.int32),
        pltpu.VMEM((128,), jnp.int32),
        pltpu.VMEM((128, HC), jnp.float32),
        pltpu.VMEM((ACCR, HC), jnp.float32),
    ],
)
def _acc_kernel(y0_hbm, y1_hbm, routed_hbm, counts_hbm, zeros_hbm, acc_out,
                pkbuf, cbuf, gidx, rows_t, acc):
    c = lax.axis_index("c")
    s = lax.axis_index("s")

    pltpu.sync_copy(zeros_hbm, acc)

    for cc, yh in ((0, y0_hbm), (1, y1_hbm)):
        @pl.when(c == cc)
        def _():
            def producer(w2, carry):
                pltpu.sync_copy(
                    routed_hbm.at[pl.ds((s * NT + w2) * SEG, SEG)],
                    pkbuf.at[pl.ds(0, SEG)])
                pltpu.sync_copy(
                    counts_hbm.at[pl.ds((s * NT + w2) * CW, CW)], cbuf)
                cnt = cbuf[pl.ds(0, 16)][0]
                nb = (cnt + 127) // 128

                def chunk(b, carry2):
                    for k in range(8):
                        vpk = pkbuf[pl.ds(b * 128 + k * 16, 16)]
                        gidx[pl.ds(k * 16, 16)] = vpk >> 10
                    pltpu.sync_copy(yh.at[gidx], rows_t)

                    def edge(e, carry3):
                        dl = pkbuf[pl.ds(b * 128 + e, 16)][0] & 1023
                        for k in range(HC // 16):
                            plsc.addupdate(
                                acc.at[dl, pl.ds(k * 16, 16)],
                                rows_t[e, pl.ds(k * 16, 16)])
                        return carry3

                    lax.fori_loop(0, 128, edge, 0)
                    return carry2

                lax.fori_loop(0, nb, chunk, 0)
                return carry

            lax.fori_loop(0, NT, producer, 0)

    pltpu.sync_copy(acc.at[pl.ds(0, OWN)],
                    acc_out.at[pl.ds(s * OWN, OWN), pl.ds(c * HC, HC)])


RB = 512
GRID = NP // RB


def _pre_body(x_ref, w_ref, p0_ref, p1_ref, y_ref, dinv_ref):
    xw = jnp.dot(x_ref[...], w_ref[...], preferred_element_type=jnp.float32)
    deg = 1.0 + p0_ref[...] + p1_ref[...]
    dinv = lax.rsqrt(deg)
    dinv_ref[...] = dinv
    y_ref[...] = xw * dinv


_pre_call = pl.pallas_call(
    _pre_body,
    grid=(GRID,),
    in_specs=[
        pl.BlockSpec((RB, IN_CH), lambda i: (i, 0)),
        pl.BlockSpec((IN_CH, IN_CH), lambda i: (0, 0)),
        pl.BlockSpec((RB, 1), lambda i: (i, 0)),
        pl.BlockSpec((RB, 1), lambda i: (i, 0)),
    ],
    out_specs=[
        pl.BlockSpec((RB, IN_CH), lambda i: (i, 0)),
        pl.BlockSpec((RB, 1), lambda i: (i, 0)),
    ],
    out_shape=[
        jax.ShapeDtypeStruct((NP, IN_CH), jnp.float32),
        jax.ShapeDtypeStruct((NP, 1), jnp.float32),
    ],
)


def _head_body(acc_ref, y_ref, dinv_ref, x_ref, bg_ref, w1_ref, b1_ref,
               w2_ref, b2_ref, w3_ref, b3_ref, x0_ref, x1_ref):
    gcn = (acc_ref[...] + y_ref[...]) * dinv_ref[...] + bg_ref[...]
    h = jnp.maximum(gcn, 0.0) + x_ref[...]
    x0 = jnp.maximum(
        jnp.dot(h, w1_ref[...], preferred_element_type=jnp.float32)
        + b1_ref[...], 0.0)
    x0_ref[...] = x0
    x1h = jnp.maximum(
        jnp.dot(x0, w2_ref[...], preferred_element_type=jnp.float32)
        + b2_ref[...], 0.0)
    x1_ref[...] = (jnp.dot(x1h, w3_ref[...],
                           preferred_element_type=jnp.float32) + b3_ref[...])


_head_call = pl.pallas_call(
    _head_body,
    grid=(GRID,),
    in_specs=[
        pl.BlockSpec((RB, IN_CH), lambda i: (i, 0)),
        pl.BlockSpec((RB, IN_CH), lambda i: (i, 0)),
        pl.BlockSpec((RB, 1), lambda i: (i, 0)),
        pl.BlockSpec((RB, IN_CH), lambda i: (i, 0)),
        pl.BlockSpec((1, IN_CH), lambda i: (0, 0)),
        pl.BlockSpec((IN_CH, MID_CH), lambda i: (0, 0)),
        pl.BlockSpec((1, MID_CH), lambda i: (0, 0)),
        pl.BlockSpec((MID_CH, MID_CH), lambda i: (0, 0)),
        pl.BlockSpec((1, MID_CH), lambda i: (0, 0)),
        pl.BlockSpec((MID_CH, 128), lambda i: (0, 0)),
        pl.BlockSpec((1, 128), lambda i: (0, 0)),
    ],
    out_specs=[
        pl.BlockSpec((RB, MID_CH), lambda i: (i, 0)),
        pl.BlockSpec((RB, 128), lambda i: (i, 0)),
    ],
    out_shape=[
        jax.ShapeDtypeStruct((NP, MID_CH), jnp.float32),
        jax.ShapeDtypeStruct((NP, 128), jnp.float32),
    ],
)


@jax.jit
def kernel(x, edge_index, W_gcn, b_gcn, W1, b1, W2, b2, W3, b3):
    ei = edge_index.astype(jnp.int32)
    src = ei[0]
    dst = ei[1]

    pad = SPAD - STRIPE
    src_e = jnp.pad(src.reshape(NT, STRIPE), ((0, 0), (0, pad)),
                    constant_values=0).reshape(-1)
    dst_e = jnp.pad(dst.reshape(NT, STRIPE), ((0, 0), (0, pad)),
                    constant_values=NP).reshape(-1)

    x_p = jnp.pad(x, ((0, NP - N), (0, 0)))

    routed, counts = _route_kernel(src_e, dst_e)

    deg_parts = _deg_kernel(routed, counts,
                            jnp.zeros((ACCR * 16,), jnp.float32))
    deg_parts = deg_parts.reshape(NC, NP, 16)
    p0 = deg_parts[0, :, 0:1]
    p1 = deg_parts[1, :, 0:1]

    y, dinv = _pre_call(x_p, W_gcn, p0, p1)

    acc = _acc_kernel(y[:, :HC], y[:, HC:], routed, counts,
                      jnp.zeros((ACCR, HC), jnp.float32))

    x0, x1 = _head_call(
        acc, y, dinv, x_p,
        b_gcn.reshape(1, IN_CH),
        W1, b1.reshape(1, MID_CH),
        W2, b2.reshape(1, MID_CH),
        jnp.pad(W3, ((0, 0), (0, 127))),
        jnp.pad(b3.reshape(1, 1), ((0, 0), (0, 127))),
    )
    return x0[:N], x1[:N, 0:1]
